# Initial kernel scaffold; baseline (speedup 1.0000x reference)
import jax
import jax.numpy as jnp
from jax.experimental import pallas as pl


def kernel(input_ids, embed_weight):
    raise NotImplementedError("write your pallas kernel here")


# speedup vs baseline: 5.8725x; 5.8725x over previous
import functools

import jax
import jax.numpy as jnp
from jax import lax
from jax.experimental import pallas as pl
from jax.experimental.pallas import tpu as pltpu
from jax.experimental.pallas import tpu_sc as plsc

VOCAB = 32000
HIDDEN = 64
B = 16384
L = 50

NC = 2
NS = 16
NW = NC * NS

SEQ_PER_W = B // NW
CHUNK_SEQ = 16
CHUNK_TOK = CHUNK_SEQ * L
NCHUNK = SEQ_PER_W // CHUNK_SEQ
IDX_MINOR = 100
IDX_ROWS = CHUNK_TOK // IDX_MINOR
NVREG = HIDDEN // 16


def _body(ids_hbm, table_hbm, lhs_hbm, pool_hbm, idx_v, rows_v, pool_v, sem):
    cid = lax.axis_index("c")
    sid = lax.axis_index("s")
    wid = sid * NC + cid

    def chunk(g, carry):
        row0 = wid * (SEQ_PER_W * L // IDX_MINOR) + g * IDX_ROWS
        tok0 = row0 * IDX_MINOR
        seq0 = wid * SEQ_PER_W + g * CHUNK_SEQ

        pltpu.sync_copy(ids_hbm.at[pl.ds(row0, IDX_ROWS)], idx_v)

        cps = [
            pltpu.async_copy(
                table_hbm.at[idx_v.at[j]],
                rows_v.at[pl.ds(j * IDX_MINOR, IDX_MINOR)],
                sem,
            )
            for j in range(IDX_ROWS)
        ]
        for cp in cps:
            cp.wait()

        pltpu.sync_copy(rows_v, lhs_hbm.at[pl.ds(tok0, CHUNK_TOK)])

        for si in range(CHUNK_SEQ):
            def jblock(jb, accs, si=si):
                base = si * L + jb * 10
                for j in range(10):
                    accs = tuple(
                        accs[cc] + rows_v[base + j, pl.ds(cc * 16, 16)]
                        for cc in range(NVREG)
                    )
                return accs
            accs = lax.fori_loop(
                0, L // 10, jblock,
                tuple(jnp.zeros((16,), jnp.float32) for _ in range(NVREG)),
            )
            for cc in range(NVREG):
                pool_v[si, pl.ds(cc * 16, 16)] = accs[cc] * (1.0 / L)
        pltpu.sync_copy(pool_v, pool_hbm.at[pl.ds(seq0, CHUNK_SEQ)])
        return carry

    lax.fori_loop(0, NCHUNK, chunk, 0)


@jax.jit
def _encode(ids2d, table):
    mesh = plsc.VectorSubcoreMesh(core_axis_name="c", subcore_axis_name="s")
    kern = functools.partial(
        pl.kernel,
        out_type=[
            jax.ShapeDtypeStruct((B * L, HIDDEN), jnp.float32),
            jax.ShapeDtypeStruct((B, HIDDEN), jnp.float32),
        ],
        mesh=mesh,
        scratch_types=[
            pltpu.VMEM((IDX_ROWS, IDX_MINOR), jnp.int32),
            pltpu.VMEM((CHUNK_TOK, HIDDEN), jnp.float32),
            pltpu.VMEM((CHUNK_SEQ, HIDDEN), jnp.float32),
            pltpu.SemaphoreType.DMA,
        ],
        compiler_params=pltpu.CompilerParams(use_tc_tiling_on_sc=False),
    )(_body)
    return kern(ids2d, table)


def kernel(input_ids, embed_weight):
    ids2d = input_ids.astype(jnp.int32).reshape(B * L // IDX_MINOR, IDX_MINOR)
    lhs_flat, pool = _encode(ids2d, embed_weight)
    return (lhs_flat.reshape(B, L, HIDDEN), pool)


# speedup vs baseline: 6.5898x; 1.1221x over previous
import functools

import jax
import jax.numpy as jnp
from jax import lax
from jax.experimental import pallas as pl
from jax.experimental.pallas import tpu as pltpu
from jax.experimental.pallas import tpu_sc as plsc

VOCAB = 32000
HIDDEN = 64
B = 16384
L = 50

NC = 2
NS = 16
NW = NC * NS

SEQ_PER_W = B // NW
CHUNK_SEQ = 16
CHUNK_TOK = CHUNK_SEQ * L
NCHUNK = SEQ_PER_W // CHUNK_SEQ
IDX_MINOR = 100
IDX_ROWS = CHUNK_TOK // IDX_MINOR
ROWS_PER_W = SEQ_PER_W * L // IDX_MINOR
NVREG = HIDDEN // 16


def _body(ids_hbm, table_hbm, lhs_hbm, pool_hbm,
          idx_v0, idx_v1, rows_v0, rows_v1, pool_v0, pool_v1,
          sem_idx, sem_g, sem_st, sem_pst):
    cid = lax.axis_index("c")
    sid = lax.axis_index("s")
    wid = sid * NC + cid

    def idx_row0(g):
        return wid * ROWS_PER_W + g * IDX_ROWS

    def fire_gathers(idx_p, rows_p):
        for j in range(IDX_ROWS):
            pltpu.async_copy(
                table_hbm.at[idx_p.at[j]],
                rows_p.at[pl.ds(j * IDX_MINOR, IDX_MINOR)],
                sem_g,
            )

    def drain_gathers(idx_p, rows_p):
        for j in range(IDX_ROWS):
            pltpu.make_async_copy(
                table_hbm.at[idx_p.at[j]],
                rows_p.at[pl.ds(j * IDX_MINOR, IDX_MINOR)],
                sem_g,
            ).wait()

    def one_iter(g, idx_p, idx_q, rows_p, rows_q, pool_p):
        tok0 = idx_row0(g) * IDX_MINOR
        seq0 = wid * SEQ_PER_W + g * CHUNK_SEQ

        @pl.when(g >= 1)
        def _():
            pltpu.make_async_copy(
                rows_q, lhs_hbm.at[pl.ds(tok0 - CHUNK_TOK, CHUNK_TOK)], sem_st
            ).wait()

        drain_gathers(idx_p, rows_p)

        pltpu.async_copy(rows_p, lhs_hbm.at[pl.ds(tok0, CHUNK_TOK)], sem_st)

        @pl.when(g + 2 < NCHUNK)
        def _():
            pltpu.async_copy(
                ids_hbm.at[pl.ds(idx_row0(g) + 2 * IDX_ROWS, IDX_ROWS)],
                idx_p, sem_idx,
            )

        @pl.when(g + 1 < NCHUNK)
        def _():
            pltpu.make_async_copy(
                ids_hbm.at[pl.ds(idx_row0(g) + IDX_ROWS, IDX_ROWS)],
                idx_q, sem_idx,
            ).wait()
            fire_gathers(idx_q, rows_q)

        @pl.when(g >= 2)
        def _():
            pltpu.make_async_copy(
                pool_p,
                pool_hbm.at[pl.ds(seq0 - 2 * CHUNK_SEQ, CHUNK_SEQ)],
                sem_pst,
            ).wait()

        for si in range(CHUNK_SEQ):
            def jblock(jb, accs, si=si):
                base = si * L + jb * 10
                for j in range(10):
                    accs = tuple(
                        accs[cc] + rows_p[base + j, pl.ds(cc * 16, 16)]
                        for cc in range(NVREG)
                    )
                return accs
            accs = lax.fori_loop(
                0, L // 10, jblock,
                tuple(jnp.zeros((16,), jnp.float32) for _ in range(NVREG)),
            )
            for cc in range(NVREG):
                pool_p[si, pl.ds(cc * 16, 16)] = accs[cc] * (1.0 / L)

        pltpu.async_copy(pool_p, pool_hbm.at[pl.ds(seq0, CHUNK_SEQ)], sem_pst)

    pltpu.async_copy(ids_hbm.at[pl.ds(idx_row0(0), IDX_ROWS)], idx_v0, sem_idx)
    pltpu.make_async_copy(
        ids_hbm.at[pl.ds(idx_row0(0), IDX_ROWS)], idx_v0, sem_idx
    ).wait()
    fire_gathers(idx_v0, rows_v0)
    pltpu.async_copy(ids_hbm.at[pl.ds(idx_row0(1), IDX_ROWS)], idx_v1, sem_idx)

    def body2(gg, carry):
        one_iter(2 * gg, idx_v0, idx_v1, rows_v0, rows_v1, pool_v0)
        one_iter(2 * gg + 1, idx_v1, idx_v0, rows_v1, rows_v0, pool_v1)
        return carry

    lax.fori_loop(0, NCHUNK // 2, body2, 0)

    last_tok0 = idx_row0(NCHUNK - 1) * IDX_MINOR
    pltpu.make_async_copy(
        rows_v1, lhs_hbm.at[pl.ds(last_tok0, CHUNK_TOK)], sem_st
    ).wait()
    seq_end = wid * SEQ_PER_W + NCHUNK * CHUNK_SEQ
    pltpu.make_async_copy(
        pool_v0, pool_hbm.at[pl.ds(seq_end - 2 * CHUNK_SEQ, CHUNK_SEQ)], sem_pst
    ).wait()
    pltpu.make_async_copy(
        pool_v1, pool_hbm.at[pl.ds(seq_end - CHUNK_SEQ, CHUNK_SEQ)], sem_pst
    ).wait()


@jax.jit
def _encode(ids2d, table):
    mesh = plsc.VectorSubcoreMesh(core_axis_name="c", subcore_axis_name="s")
    kern = functools.partial(
        pl.kernel,
        out_type=[
            jax.ShapeDtypeStruct((B * L, HIDDEN), jnp.float32),
            jax.ShapeDtypeStruct((B, HIDDEN), jnp.float32),
        ],
        mesh=mesh,
        scratch_types=[
            pltpu.VMEM((IDX_ROWS, IDX_MINOR), jnp.int32),
            pltpu.VMEM((IDX_ROWS, IDX_MINOR), jnp.int32),
            pltpu.VMEM((CHUNK_TOK, HIDDEN), jnp.float32),
            pltpu.VMEM((CHUNK_TOK, HIDDEN), jnp.float32),
            pltpu.VMEM((CHUNK_SEQ, HIDDEN), jnp.float32),
            pltpu.VMEM((CHUNK_SEQ, HIDDEN), jnp.float32),
            pltpu.SemaphoreType.DMA,
            pltpu.SemaphoreType.DMA,
            pltpu.SemaphoreType.DMA,
            pltpu.SemaphoreType.DMA,
        ],
        compiler_params=pltpu.CompilerParams(use_tc_tiling_on_sc=False),
    )(_body)
    return kern(ids2d, table)


def kernel(input_ids, embed_weight):
    ids2d = input_ids.astype(jnp.int32).reshape(B * L // IDX_MINOR, IDX_MINOR)
    lhs_flat, pool = _encode(ids2d, embed_weight)
    return (lhs_flat.reshape(B, L, HIDDEN), pool)
